# baseline (speedup 1.0000x reference)
import functools

import jax
import jax.numpy as jnp
import numpy as np
from jax import lax
from jax.experimental import pallas as pl
from jax.experimental.pallas import tpu as pltpu
from jax.experimental.pallas import tpu_sc as plsc

VOCAB = 100000
EMB = 16
BATCH = 1024
CTX = 20

_NC = 2
_NS = 16
_NW = _NC * _NS
_S_PER_W = BATCH // _NW
_IDX_PER_W = _S_PER_W * CTX
_GCHUNK = 128
_NCHUNK = _IDX_PER_W // _GCHUNK
_RPP = 128 // EMB
_ACC_ROWS = _S_PER_W * _RPP

_SMAP = np.asarray(
    (np.arange(_IDX_PER_W) // CTX * _RPP).reshape(_NCHUNK, _GCHUNK),
    dtype=np.int32)


@functools.cache
def _make_sc_embed_mean():
    mesh = plsc.VectorSubcoreMesh(core_axis_name="c", subcore_axis_name="s")

    @functools.partial(
        pl.kernel,
        mesh=mesh,
        out_type=jax.ShapeDtypeStruct((BATCH * EMB,), jnp.float32),
        scratch_types=[
            pltpu.VMEM((_NCHUNK, _GCHUNK), jnp.int32),
            pltpu.VMEM((_NCHUNK, _GCHUNK), jnp.int32),
            pltpu.VMEM((_NCHUNK, _GCHUNK), jnp.int32),
            pltpu.VMEM((_NCHUNK, _GCHUNK), jnp.int32),
            pltpu.VMEM((_IDX_PER_W, 128), jnp.float32),
            pltpu.VMEM_SHARED((_NS * _ACC_ROWS, 128), jnp.float32),
            pltpu.VMEM((_S_PER_W * EMB,), jnp.float32),
            pltpu.SemaphoreType.DMA,
            pltpu.SemaphoreType.DMA,
        ],
    )
    def sc_embed_mean(idx_hbm, smap_hbm, table_hbm, out_hbm, idx_v, smap_v,
                      idxd_v, sidx_v, rows_v, acc_sh, avg_v, gsem, ssem):
        wid = lax.axis_index("s") * _NC + lax.axis_index("c")
        sid = lax.axis_index("s")
        abase = sid * _ACC_ROWS

        zeros16 = jnp.zeros((16,), jnp.float32)
        for s in range(_S_PER_W):
            for m in range(_RPP):
                rows_v[s * _RPP + m, pl.ds(m * EMB, EMB)] = zeros16
        zpush = pltpu.async_copy(
            rows_v.at[pl.ds(0, _ACC_ROWS)],
            acc_sh.at[pl.ds(abase, _ACC_ROWS)], ssem)

        pltpu.sync_copy(idx_hbm.at[wid], idx_v)
        pltpu.sync_copy(smap_hbm, smap_v)

        gathers = []
        for k in range(_NCHUNK):
            for q in range(_GCHUNK // 16):
                sl = pl.ds(q * 16, 16)
                v = idx_v[k, sl]
                idxd_v[k, sl] = lax.shift_right_logical(v, 3)
                sidx_v[k, sl] = smap_v[k, sl] + (v & (_RPP - 1)) + abase
            gathers.append(pltpu.async_copy(
                table_hbm.at[idxd_v.at[k]],
                rows_v.at[pl.ds(k * _GCHUNK, _GCHUNK)],
                gsem,
            ))
        zpush.wait()
        for g in gathers:
            g.wait()

        scatters = [
            pltpu.async_copy(
                rows_v.at[pl.ds(k * _GCHUNK, _GCHUNK)],
                acc_sh.at[sidx_v.at[k]],
                ssem,
                add=True,
            )
            for k in range(_NCHUNK)
        ]
        for s in scatters:
            s.wait()

        pltpu.sync_copy(acc_sh.at[pl.ds(abase, _ACC_ROWS)],
                        rows_v.at[pl.ds(0, _ACC_ROWS)])
        inv = jnp.float32(1.0 / CTX)
        for s in range(_S_PER_W):
            acc = rows_v[s * _RPP + 0, pl.ds(0, EMB)]
            for m in range(1, _RPP):
                acc = acc + rows_v[s * _RPP + m, pl.ds(m * EMB, EMB)]
            avg_v[pl.ds(s * EMB, EMB)] = acc * inv

        pltpu.sync_copy(
            avg_v,
            out_hbm.at[pl.ds(wid * _S_PER_W * EMB, _S_PER_W * EMB)])

    return sc_embed_mean


_VT = 2048
_GRID = (VOCAB + _VT - 1) // _VT


def _proj_body(w_ref, avg_ref, b_ref, out_ref):
    out_ref[...] = (
        lax.dot_general(
            w_ref[...], avg_ref[...],
            dimension_numbers=(((0,), (1,)), ((), ())),
            preferred_element_type=jnp.float32,
        )
        + b_ref[...]
    )


def _tc_project_t(avg, W, bcol):
    return pl.pallas_call(
        _proj_body,
        grid=(_GRID,),
        in_specs=[
            pl.BlockSpec((EMB, _VT), lambda j: (0, j)),
            pl.BlockSpec((BATCH, EMB), lambda j: (0, 0)),
            pl.BlockSpec((_VT, 1), lambda j: (j, 0)),
        ],
        out_specs=pl.BlockSpec((_VT, BATCH), lambda j: (j, 0)),
        out_shape=jax.ShapeDtypeStruct((VOCAB, BATCH), jnp.float32),
    )(W, avg, bcol)


def kernel(inputs, emb_table, W, b):
    idx3 = inputs.astype(jnp.int32).reshape(_NW, _NCHUNK, _GCHUNK)
    smap = jnp.asarray(_SMAP)
    table128 = emb_table.reshape(VOCAB // _RPP, 128)
    avg = _make_sc_embed_mean()(idx3, smap, table128).reshape(BATCH, EMB)
    return _tc_project_t(avg, W, b.reshape(VOCAB, 1)).T


# speedup vs baseline: 1.1944x; 1.1944x over previous
import functools

import jax
import jax.numpy as jnp
import numpy as np
from jax import lax
from jax.experimental import pallas as pl
from jax.experimental.pallas import tpu as pltpu
from jax.experimental.pallas import tpu_sc as plsc

VOCAB = 100000
EMB = 16
BATCH = 1024
CTX = 20

_NC = 2
_NS = 16
_NW = _NC * _NS
_S_PER_W = BATCH // _NW
_IDX_PER_W = _S_PER_W * CTX
_GCHUNK = 128
_NCHUNK = _IDX_PER_W // _GCHUNK
_RPP = 128 // EMB
_ACC_ROWS = _S_PER_W * _RPP

_SMAP = np.asarray(
    (np.arange(_IDX_PER_W) // CTX * _RPP).reshape(_NCHUNK, _GCHUNK),
    dtype=np.int32)


@functools.cache
def _make_sc_embed_mean():
    mesh = plsc.VectorSubcoreMesh(core_axis_name="c", subcore_axis_name="s")

    @functools.partial(
        pl.kernel,
        mesh=mesh,
        out_type=jax.ShapeDtypeStruct((BATCH * EMB,), jnp.float32),
        scratch_types=[
            pltpu.VMEM((_NCHUNK, _GCHUNK), jnp.int32),
            pltpu.VMEM((_NCHUNK, _GCHUNK), jnp.int32),
            pltpu.VMEM((_NCHUNK, _GCHUNK), jnp.int32),
            pltpu.VMEM((_NCHUNK, _GCHUNK), jnp.int32),
            pltpu.VMEM((_IDX_PER_W, 128), jnp.float32),
            pltpu.VMEM_SHARED((_NS * _ACC_ROWS, 128), jnp.float32),
            pltpu.VMEM((_S_PER_W * EMB,), jnp.float32),
            pltpu.SemaphoreType.DMA,
            pltpu.SemaphoreType.DMA,
        ],
    )
    def sc_embed_mean(idx_hbm, smap_hbm, table_hbm, out_hbm, idx_v, smap_v,
                      idxd_v, sidx_v, rows_v, acc_sh, avg_v, gsem, ssem):
        wid = lax.axis_index("s") * _NC + lax.axis_index("c")
        sid = lax.axis_index("s")
        abase = sid * _ACC_ROWS

        zeros16 = jnp.zeros((16,), jnp.float32)
        for s in range(_S_PER_W):
            for m in range(_RPP):
                rows_v[s * _RPP + m, pl.ds(m * EMB, EMB)] = zeros16
        zpush = pltpu.async_copy(
            rows_v.at[pl.ds(0, _ACC_ROWS)],
            acc_sh.at[pl.ds(abase, _ACC_ROWS)], ssem)

        pltpu.sync_copy(idx_hbm.at[wid], idx_v)
        pltpu.sync_copy(smap_hbm, smap_v)

        gathers = []
        for k in range(_NCHUNK):
            for q in range(_GCHUNK // 16):
                sl = pl.ds(q * 16, 16)
                v = idx_v[k, sl]
                idxd_v[k, sl] = lax.shift_right_logical(v, 3)
                sidx_v[k, sl] = smap_v[k, sl] + (v & (_RPP - 1)) + abase
            gathers.append(pltpu.async_copy(
                table_hbm.at[idxd_v.at[k]],
                rows_v.at[pl.ds(k * _GCHUNK, _GCHUNK)],
                gsem,
            ))
        zpush.wait()
        for g in gathers:
            g.wait()

        scatters = [
            pltpu.async_copy(
                rows_v.at[pl.ds(k * _GCHUNK, _GCHUNK)],
                acc_sh.at[sidx_v.at[k]],
                ssem,
                add=True,
            )
            for k in range(_NCHUNK)
        ]
        for s in scatters:
            s.wait()

        pltpu.sync_copy(acc_sh.at[pl.ds(abase, _ACC_ROWS)],
                        rows_v.at[pl.ds(0, _ACC_ROWS)])
        inv = jnp.float32(1.0 / CTX)
        for s in range(_S_PER_W):
            acc = rows_v[s * _RPP + 0, pl.ds(0, EMB)]
            for m in range(1, _RPP):
                acc = acc + rows_v[s * _RPP + m, pl.ds(m * EMB, EMB)]
            avg_v[pl.ds(s * EMB, EMB)] = acc * inv

        pltpu.sync_copy(
            avg_v,
            out_hbm.at[pl.ds(wid * _S_PER_W * EMB, _S_PER_W * EMB)])

    return sc_embed_mean


_VT = 2048
_GRID = (VOCAB + _VT - 1) // _VT


def _proj_body(w_ref, avg_ref, out_ref):
    out_ref[...] = lax.dot_general(
        w_ref[...], avg_ref[...],
        dimension_numbers=(((0,), (1,)), ((), ())),
        preferred_element_type=jnp.float32,
    )


def _tc_project_t(avg_aug, w_aug):
    return pl.pallas_call(
        _proj_body,
        grid=(_GRID,),
        in_specs=[
            pl.BlockSpec((EMB + 1, _VT), lambda j: (0, j)),
            pl.BlockSpec((BATCH, EMB + 1), lambda j: (0, 0)),
        ],
        out_specs=pl.BlockSpec((_VT, BATCH), lambda j: (j, 0)),
        out_shape=jax.ShapeDtypeStruct((VOCAB, BATCH), jnp.float32),
    )(w_aug, avg_aug)


def kernel(inputs, emb_table, W, b):
    idx3 = inputs.astype(jnp.int32).reshape(_NW, _NCHUNK, _GCHUNK)
    smap = jnp.asarray(_SMAP)
    table128 = emb_table.reshape(VOCAB // _RPP, 128)
    avg = _make_sc_embed_mean()(idx3, smap, table128).reshape(BATCH, EMB)
    w_aug = jnp.concatenate([W, b[None, :]], axis=0)
    avg_aug = jnp.concatenate(
        [avg, jnp.ones((BATCH, 1), jnp.float32)], axis=1)
    return _tc_project_t(avg_aug, w_aug).T


# speedup vs baseline: 1.2287x; 1.0287x over previous
import functools

import jax
import jax.numpy as jnp
import numpy as np
from jax import lax
from jax.experimental import pallas as pl
from jax.experimental.pallas import tpu as pltpu
from jax.experimental.pallas import tpu_sc as plsc

VOCAB = 100000
EMB = 16
BATCH = 1024
CTX = 20

_NC = 2
_NS = 16
_NW = _NC * _NS
_S_PER_W = BATCH // _NW
_IDX_PER_W = _S_PER_W * CTX
_GCHUNK = 128
_NCHUNK = _IDX_PER_W // _GCHUNK

_SMAP = np.asarray(
    (np.arange(_IDX_PER_W) // CTX).reshape(_NCHUNK, _GCHUNK),
    dtype=np.int32)


@functools.cache
def _make_sc_embed_mean():
    mesh = plsc.VectorSubcoreMesh(core_axis_name="c", subcore_axis_name="s")

    @functools.partial(
        pl.kernel,
        mesh=mesh,
        out_type=jax.ShapeDtypeStruct((BATCH * EMB,), jnp.float32),
        scratch_types=[
            pltpu.VMEM((_NCHUNK, _GCHUNK), jnp.int32),
            pltpu.VMEM((_NCHUNK, _GCHUNK), jnp.int32),
            pltpu.VMEM((_NCHUNK, _GCHUNK), jnp.int32),
            pltpu.VMEM((_IDX_PER_W, 128), jnp.float32),
            pltpu.VMEM_SHARED((_NS * _S_PER_W, 128), jnp.float32),
            pltpu.VMEM((_S_PER_W * EMB,), jnp.float32),
            pltpu.SemaphoreType.DMA,
            pltpu.SemaphoreType.DMA,
        ],
    )
    def sc_embed_mean(idx_hbm, smap_hbm, table_hbm, out_hbm, idx_v, smap_v,
                      sidx_v, rows_v, acc_sh, avg_v, gsem, ssem):
        wid = lax.axis_index("s") * _NC + lax.axis_index("c")
        sid = lax.axis_index("s")
        abase = sid * _S_PER_W

        zeros16 = jnp.zeros((16,), jnp.float32)
        for s in range(_S_PER_W):
            rows_v[s, pl.ds(0, EMB)] = zeros16
        zpush = pltpu.async_copy(
            rows_v.at[pl.ds(0, _S_PER_W)],
            acc_sh.at[pl.ds(abase, _S_PER_W)], ssem)

        pltpu.sync_copy(idx_hbm.at[wid], idx_v)
        pltpu.sync_copy(smap_hbm, smap_v)

        gathers = []
        for k in range(_NCHUNK):
            for q in range(_GCHUNK // 16):
                sl = pl.ds(q * 16, 16)
                sidx_v[k, sl] = smap_v[k, sl] + abase
            gathers.append(pltpu.async_copy(
                table_hbm.at[idx_v.at[k]],
                rows_v.at[pl.ds(k * _GCHUNK, _GCHUNK)],
                gsem,
            ))
        zpush.wait()
        for g in gathers:
            g.wait()

        scatters = [
            pltpu.async_copy(
                rows_v.at[pl.ds(k * _GCHUNK, _GCHUNK)],
                acc_sh.at[sidx_v.at[k]],
                ssem,
                add=True,
            )
            for k in range(_NCHUNK)
        ]
        for s in scatters:
            s.wait()

        pltpu.sync_copy(acc_sh.at[pl.ds(abase, _S_PER_W)],
                        rows_v.at[pl.ds(0, _S_PER_W)])
        inv = jnp.float32(1.0 / CTX)
        for s in range(_S_PER_W):
            avg_v[pl.ds(s * EMB, EMB)] = rows_v[s, pl.ds(0, EMB)] * inv

        pltpu.sync_copy(
            avg_v,
            out_hbm.at[pl.ds(wid * _S_PER_W * EMB, _S_PER_W * EMB)])

    return sc_embed_mean


_VT = 2048
_GRID = (VOCAB + _VT - 1) // _VT


def _proj_body(w_ref, avg_ref, out_ref):
    out_ref[...] = lax.dot_general(
        w_ref[...], avg_ref[...],
        dimension_numbers=(((0,), (1,)), ((), ())),
        preferred_element_type=jnp.float32,
    )


def _tc_project_t(avg_aug, w_aug):
    return pl.pallas_call(
        _proj_body,
        grid=(_GRID,),
        in_specs=[
            pl.BlockSpec((EMB + 1, _VT), lambda j: (0, j)),
            pl.BlockSpec((BATCH, EMB + 1), lambda j: (0, 0)),
        ],
        out_specs=pl.BlockSpec((_VT, BATCH), lambda j: (j, 0)),
        out_shape=jax.ShapeDtypeStruct((VOCAB, BATCH), jnp.float32),
    )(w_aug, avg_aug)


def kernel(inputs, emb_table, W, b):
    idx3 = inputs.astype(jnp.int32).reshape(_NW, _NCHUNK, _GCHUNK)
    smap = jnp.asarray(_SMAP)
    table_pad = jnp.pad(emb_table, ((0, 0), (0, 128 - EMB)))
    avg = _make_sc_embed_mean()(idx3, smap, table_pad).reshape(BATCH, EMB)
    w_aug = jnp.concatenate([W, b[None, :]], axis=0)
    avg_aug = jnp.concatenate(
        [avg, jnp.ones((BATCH, 1), jnp.float32)], axis=1)
    return _tc_project_t(avg_aug, w_aug).T


# speedup vs baseline: 1.2598x; 1.0253x over previous
import functools

import jax
import jax.numpy as jnp
import numpy as np
from jax import lax
from jax.experimental import pallas as pl
from jax.experimental.pallas import tpu as pltpu
from jax.experimental.pallas import tpu_sc as plsc

VOCAB = 100000
EMB = 16
BATCH = 1024
CTX = 20

_NC = 2
_NS = 16
_NW = _NC * _NS
_S_PER_W = BATCH // _NW
_IDX_PER_W = _S_PER_W * CTX
_GCHUNK = 128
_NCHUNK = _IDX_PER_W // _GCHUNK

_SMAP = np.asarray(
    (np.arange(_IDX_PER_W) // CTX).reshape(_NCHUNK, _GCHUNK),
    dtype=np.int32)


@functools.cache
def _make_sc_embed_mean():
    mesh = plsc.VectorSubcoreMesh(core_axis_name="c", subcore_axis_name="s")

    @functools.partial(
        pl.kernel,
        mesh=mesh,
        out_type=jax.ShapeDtypeStruct((BATCH * EMB,), jnp.float32),
        compiler_params=pltpu.CompilerParams(use_tc_tiling_on_sc=False),
        scratch_types=[
            pltpu.VMEM((_NCHUNK, _GCHUNK), jnp.int32),
            pltpu.VMEM((_NCHUNK, _GCHUNK), jnp.int32),
            pltpu.VMEM((_NCHUNK, _GCHUNK), jnp.int32),
            pltpu.VMEM((_IDX_PER_W, EMB), jnp.float32),
            pltpu.VMEM_SHARED((_NS * _S_PER_W, EMB), jnp.float32),
            pltpu.VMEM((_S_PER_W * EMB,), jnp.float32),
            pltpu.SemaphoreType.DMA,
            pltpu.SemaphoreType.DMA,
        ],
    )
    def sc_embed_mean(idx_hbm, smap_hbm, table_hbm, out_hbm, idx_v, smap_v,
                      sidx_v, rows_v, acc_sh, avg_v, gsem, ssem):
        wid = lax.axis_index("s") * _NC + lax.axis_index("c")
        sid = lax.axis_index("s")
        abase = sid * _S_PER_W

        zeros16 = jnp.zeros((16,), jnp.float32)
        for s in range(_S_PER_W):
            rows_v[s] = zeros16
        zpush = pltpu.async_copy(
            rows_v.at[pl.ds(0, _S_PER_W)],
            acc_sh.at[pl.ds(abase, _S_PER_W)], ssem)

        pltpu.sync_copy(idx_hbm.at[wid], idx_v)
        pltpu.sync_copy(smap_hbm, smap_v)

        gathers = []
        for k in range(_NCHUNK):
            for q in range(_GCHUNK // 16):
                sl = pl.ds(q * 16, 16)
                sidx_v[k, sl] = smap_v[k, sl] + abase
            gathers.append(pltpu.async_copy(
                table_hbm.at[idx_v.at[k]],
                rows_v.at[pl.ds(k * _GCHUNK, _GCHUNK)],
                gsem,
            ))
        zpush.wait()
        for g in gathers:
            g.wait()

        scatters = [
            pltpu.async_copy(
                rows_v.at[pl.ds(k * _GCHUNK, _GCHUNK)],
                acc_sh.at[sidx_v.at[k]],
                ssem,
                add=True,
            )
            for k in range(_NCHUNK)
        ]
        for s in scatters:
            s.wait()

        pltpu.sync_copy(acc_sh.at[pl.ds(abase, _S_PER_W)],
                        rows_v.at[pl.ds(0, _S_PER_W)])
        inv = jnp.float32(1.0 / CTX)
        for s in range(_S_PER_W):
            avg_v[pl.ds(s * EMB, EMB)] = rows_v[s] * inv

        pltpu.sync_copy(
            avg_v,
            out_hbm.at[pl.ds(wid * _S_PER_W * EMB, _S_PER_W * EMB)])

    return sc_embed_mean


_VT = 2048
_GRID = (VOCAB + _VT - 1) // _VT


def _proj_body(w_ref, avg_ref, out_ref):
    out_ref[...] = lax.dot_general(
        w_ref[...], avg_ref[...],
        dimension_numbers=(((0,), (1,)), ((), ())),
        preferred_element_type=jnp.float32,
    )


def _tc_project_t(avg_aug, w_aug):
    return pl.pallas_call(
        _proj_body,
        grid=(_GRID,),
        in_specs=[
            pl.BlockSpec((EMB + 1, _VT), lambda j: (0, j)),
            pl.BlockSpec((BATCH, EMB + 1), lambda j: (0, 0)),
        ],
        out_specs=pl.BlockSpec((_VT, BATCH), lambda j: (j, 0)),
        out_shape=jax.ShapeDtypeStruct((VOCAB, BATCH), jnp.float32),
    )(w_aug, avg_aug)


def kernel(inputs, emb_table, W, b):
    idx3 = inputs.astype(jnp.int32).reshape(_NW, _NCHUNK, _GCHUNK)
    smap = jnp.asarray(_SMAP)
    avg = _make_sc_embed_mean()(idx3, smap, emb_table).reshape(BATCH, EMB)
    w_aug = jnp.concatenate([W, b[None, :]], axis=0)
    avg_aug = jnp.concatenate(
        [avg, jnp.ones((BATCH, 1), jnp.float32)], axis=1)
    return _tc_project_t(avg_aug, w_aug).T
